# Initial kernel scaffold; baseline (speedup 1.0000x reference)
import jax
import jax.numpy as jnp
from jax.experimental import pallas as pl


def kernel(positions, hidden_states, w_qkv, w_o, q_norm_w, k_norm_w, ln1_w, ln2_w, gate_w, w_gate_up, w_down):
    raise NotImplementedError("write your pallas kernel here")


# speedup vs baseline: 1.3142x; 1.3142x over previous
import functools
import jax
import jax.numpy as jnp
from jax import lax
from jax.experimental import pallas as pl
from jax.experimental.pallas import tpu as pltpu

T = 2048
D = 1024
H = 16
KV = 4
HD = 64
E = 8
K = 2
FF = 1024
EPS = 1e-6
THETA = 10000.0

BT = 256
NT = T // BT
HALF = HD // 2


def _rms(x, w, eps=EPS):
    var = jnp.mean(x * x, axis=-1, keepdims=True)
    return x * lax.rsqrt(var + eps) * w


def _pre_attn_kernel(x_ref, wqkv_ref, ln1_ref, qn_ref, kn_ref, pos_ref,
                     q_ref, k_ref, v_ref):
    x = x_ref[...]
    h = _rms(x, ln1_ref[...])
    qkv = jnp.dot(h, wqkv_ref[...], preferred_element_type=jnp.float32)

    pos = pos_ref[...].astype(jnp.float32)
    i2 = lax.broadcasted_iota(jnp.int32, (1, HALF), 1).astype(jnp.float32)
    inv_freq = jnp.exp(i2 * (-2.0 * jnp.log(THETA) / HD))
    freqs = pos * inv_freq
    cos = jnp.cos(freqs)
    sin = jnp.sin(freqs)

    def rope_norm(t, w):
        t = _rms(t, w)
        t1 = t[:, :HALF]
        t2 = t[:, HALF:]
        return jnp.concatenate([t1 * cos - t2 * sin, t2 * cos + t1 * sin], axis=1)

    qn = qn_ref[...]
    kn = kn_ref[...]
    for hh in range(H):
        q_ref[hh, :, :] = rope_norm(qkv[:, hh * HD:(hh + 1) * HD], qn)
    for hh in range(KV):
        base = H * HD + hh * HD
        k_ref[hh, :, :] = rope_norm(qkv[:, base:base + HD], kn)
        v_ref[hh, :, :] = qkv[:, H * HD + KV * HD + hh * HD:
                              H * HD + KV * HD + (hh + 1) * HD]


def _pre_attn(x, w_qkv, ln1_w, q_norm_w, k_norm_w, positions):
    return pl.pallas_call(
        _pre_attn_kernel,
        grid=(NT,),
        in_specs=[
            pl.BlockSpec((BT, D), lambda i: (i, 0)),
            pl.BlockSpec((D, (H + 2 * KV) * HD), lambda i: (0, 0)),
            pl.BlockSpec((1, D), lambda i: (0, 0)),
            pl.BlockSpec((1, HD), lambda i: (0, 0)),
            pl.BlockSpec((1, HD), lambda i: (0, 0)),
            pl.BlockSpec((BT, 1), lambda i: (i, 0)),
        ],
        out_specs=[
            pl.BlockSpec((H, BT, HD), lambda i: (0, i, 0)),
            pl.BlockSpec((KV, BT, HD), lambda i: (0, i, 0)),
            pl.BlockSpec((KV, BT, HD), lambda i: (0, i, 0)),
        ],
        out_shape=[
            jax.ShapeDtypeStruct((H, T, HD), jnp.float32),
            jax.ShapeDtypeStruct((KV, T, HD), jnp.float32),
            jax.ShapeDtypeStruct((KV, T, HD), jnp.float32),
        ],
    )(x, w_qkv, ln1_w.reshape(1, D), q_norm_w.reshape(1, HD),
      k_norm_w.reshape(1, HD), positions.reshape(T, 1))


def _attn_kernel(q_ref, k_ref, v_ref, o_ref):
    iq = pl.program_id(1)
    q = q_ref[0]
    ks = k_ref[0]
    vs = v_ref[0]
    s = lax.dot_general(q, ks, (((1,), (1,)), ((), ())),
                        preferred_element_type=jnp.float32)
    s = s * (HD ** -0.5)
    row = iq * BT + lax.broadcasted_iota(jnp.int32, (BT, T), 0)
    col = lax.broadcasted_iota(jnp.int32, (BT, T), 1)
    s = jnp.where(row >= col, s, jnp.float32(-1e30))
    m = jnp.max(s, axis=1, keepdims=True)
    p = jnp.exp(s - m)
    p = p / jnp.sum(p, axis=1, keepdims=True)
    o_ref[0] = jnp.dot(p, vs, preferred_element_type=jnp.float32)


def _attention(q3d, k3d, v3d):
    return pl.pallas_call(
        _attn_kernel,
        grid=(H, NT),
        in_specs=[
            pl.BlockSpec((1, BT, HD), lambda hh, i: (hh, i, 0)),
            pl.BlockSpec((1, T, HD), lambda hh, i: (hh // (H // KV), 0, 0)),
            pl.BlockSpec((1, T, HD), lambda hh, i: (hh // (H // KV), 0, 0)),
        ],
        out_specs=pl.BlockSpec((1, BT, HD), lambda hh, i: (hh, i, 0)),
        out_shape=jax.ShapeDtypeStruct((H, T, HD), jnp.float32),
    )(q3d, k3d, v3d)


def _post_attn_kernel(o_ref, res_ref, wo_ref, ln2_ref, gw_ref,
                      x1_ref, h2_ref, lg_ref):
    o2d = jnp.concatenate([o_ref[hh] for hh in range(H)], axis=1)
    x1 = res_ref[...] + jnp.dot(o2d, wo_ref[...],
                                preferred_element_type=jnp.float32)
    h2 = _rms(x1, ln2_ref[...])
    x1_ref[...] = x1
    h2_ref[...] = h2
    lg_ref[...] = jnp.dot(h2, gw_ref[...], preferred_element_type=jnp.float32)


def _post_attn(o2d, res, w_o, ln2_w, gate_w):
    return pl.pallas_call(
        _post_attn_kernel,
        grid=(NT,),
        in_specs=[
            pl.BlockSpec((H, BT, HD), lambda i: (0, i, 0)),
            pl.BlockSpec((BT, D), lambda i: (i, 0)),
            pl.BlockSpec((H * HD, D), lambda i: (0, 0)),
            pl.BlockSpec((1, D), lambda i: (0, 0)),
            pl.BlockSpec((D, E), lambda i: (0, 0)),
        ],
        out_specs=[
            pl.BlockSpec((BT, D), lambda i: (i, 0)),
            pl.BlockSpec((BT, D), lambda i: (i, 0)),
            pl.BlockSpec((BT, E), lambda i: (i, 0)),
        ],
        out_shape=[
            jax.ShapeDtypeStruct((T, D), jnp.float32),
            jax.ShapeDtypeStruct((T, D), jnp.float32),
            jax.ShapeDtypeStruct((T, E), jnp.float32),
        ],
    )(o2d, res, w_o, ln2_w.reshape(1, D), gate_w)


def _router_kernel(lg_ref, comb_ref):
    lg = lg_ref[...]
    m = jnp.max(lg, axis=1, keepdims=True)
    p = jnp.exp(lg - m)
    p = p / jnp.sum(p, axis=1, keepdims=True)
    ii = lax.broadcasted_iota(jnp.int32, (BT, E), 1)
    m1 = jnp.max(p, axis=1, keepdims=True)
    i1 = jnp.min(jnp.where(p == m1, ii, E), axis=1, keepdims=True)
    p2 = jnp.where(ii == i1, -1.0, p)
    m2 = jnp.max(p2, axis=1, keepdims=True)
    i2 = jnp.min(jnp.where(p2 == m2, ii, E), axis=1, keepdims=True)
    s = m1 + m2
    comb_ref[...] = jnp.where(ii == i1, m1 / s, 0.0) + jnp.where(ii == i2, m2 / s, 0.0)


def _router(logits):
    return pl.pallas_call(
        _router_kernel,
        grid=(NT,),
        in_specs=[pl.BlockSpec((BT, E), lambda i: (i, 0))],
        out_specs=pl.BlockSpec((BT, E), lambda i: (i, 0)),
        out_shape=jax.ShapeDtypeStruct((T, E), jnp.float32),
    )(logits)


def _moe_kernel(h2_ref, x1_ref, wgu_ref, wd_ref, comb_ref, out_ref):
    e = pl.program_id(1)
    h2 = h2_ref[...]
    gu = jnp.dot(h2, wgu_ref[0], preferred_element_type=jnp.float32)
    g = gu[:, :FF]
    u = gu[:, FF:]
    act = g * (1.0 / (1.0 + jnp.exp(-g))) * u
    y = jnp.dot(act, wd_ref[0], preferred_element_type=jnp.float32)
    sel = (lax.broadcasted_iota(jnp.int32, (1, E), 1) == e).astype(jnp.float32)
    y = y * jnp.sum(comb_ref[...] * sel, axis=1, keepdims=True)

    @pl.when(e == 0)
    def _():
        out_ref[...] = x1_ref[...] + y

    @pl.when(e != 0)
    def _():
        out_ref[...] += y


BTM = 1024
NTM = T // BTM


def _moe(h2, x1, w_gate_up, w_down, combine):
    return pl.pallas_call(
        _moe_kernel,
        grid=(NTM, E),
        in_specs=[
            pl.BlockSpec((BTM, D), lambda it, e: (it, 0)),
            pl.BlockSpec((BTM, D), lambda it, e: (it, 0)),
            pl.BlockSpec((1, D, 2 * FF), lambda it, e: (e, 0, 0)),
            pl.BlockSpec((1, FF, D), lambda it, e: (e, 0, 0)),
            pl.BlockSpec((BTM, E), lambda it, e: (it, 0)),
        ],
        out_specs=pl.BlockSpec((BTM, D), lambda it, e: (it, 0)),
        out_shape=jax.ShapeDtypeStruct((T, D), jnp.float32),
    )(h2, x1, w_gate_up, w_down, combine)


@jax.jit
def _layer(positions, hidden_states, w_qkv, w_o, q_norm_w, k_norm_w,
           ln1_w, ln2_w, gate_w, w_gate_up, w_down):
    q2d, k2d, v2d = _pre_attn(hidden_states, w_qkv, ln1_w, q_norm_w,
                              k_norm_w, positions)
    o2d = _attention(q2d, k2d, v2d)
    x1, h2, logits = _post_attn(o2d, hidden_states, w_o, ln2_w, gate_w)
    combine = _router(logits)
    return _moe(h2, x1, w_gate_up, w_down, combine)


def kernel(positions, hidden_states, w_qkv, w_o, q_norm_w, k_norm_w,
           ln1_w, ln2_w, gate_w, w_gate_up, w_down):
    return _layer(positions, hidden_states, w_qkv, w_o, q_norm_w, k_norm_w,
                  ln1_w, ln2_w, gate_w, w_gate_up, w_down)
